# Initial kernel scaffold; baseline (speedup 1.0000x reference)
import functools

import jax
import jax.numpy as jnp
from jax import lax
from jax.experimental import pallas as pl
from jax.experimental.pallas import tpu as pltpu
from jax.experimental.pallas import tpu_sc as plsc

N_KG = 2000
NUM_RELS = 100
MAX_VISIT = 20
HID = 128
LAYERS = 3
DECAY = 0.01
B = 32
N = 10000
E = 160000

NC, NS = 2, 16
NW = NC * NS
NPW = 320
NP = NW * NPW
EPW = 5120
EP = NW * EPW
AGG_ROWS = 10048
F32 = jnp.float32
I32 = jnp.int32


def _k0_body(ne_ref, re_ref, lw_ref, lb_ref, wrw_ref, wrb_ref, ehr_ref,
             xproj_o, relmsg_o, xnode_o):
    lw = lw_ref[...]
    lb = lb_ref[...][None, :]
    xproj = jnp.dot(ne_ref[...], lw, preferred_element_type=F32) + lb
    relp = jnp.dot(re_ref[...], lw, preferred_element_type=F32) + lb
    for l in range(LAYERS):
        w = jnp.sum(relp * wrw_ref[l][None, :], axis=1, keepdims=True) + wrb_ref[l]
        relmsg_o[l] = w * relp
    s = jnp.sum(ehr_ref[...], axis=1, keepdims=True)
    xn = jnp.dot(ehr_ref[...], ne_ref[...], preferred_element_type=F32) / s
    xnode_o[...] = jnp.dot(xn, lw, preferred_element_type=F32) + lb
    xproj_o[...] = xproj


def _k0(node_emb, rel_emb, lin_W, lin_b, wr_Wr, wr_br, ehr):
    return pl.pallas_call(
        _k0_body,
        out_shape=(
            jax.ShapeDtypeStruct((N_KG, HID), F32),
            jax.ShapeDtypeStruct((LAYERS, NUM_RELS, HID), F32),
            jax.ShapeDtypeStruct((B, HID), F32),
        ),
    )(node_emb, rel_emb, lin_W, lin_b, wr_Wr, wr_br, ehr)


_KGT = 500


def _k1_body(vn_ref, aw_ref, ab_ref, bw_ref, bb_ref, attn_o):
    vn2 = vn_ref[...].reshape(B * MAX_VISIT, N_KG)
    logits = jnp.dot(vn2, aw_ref[0], preferred_element_type=F32) + ab_ref[...]
    logits = logits.reshape(B, MAX_VISIT, _KGT)
    m = jnp.max(logits, axis=1, keepdims=True)
    ex = jnp.exp(logits - m)
    alpha = ex / jnp.sum(ex, axis=1, keepdims=True)
    bet = jnp.tanh(jnp.sum(vn2 * bw_ref[0], axis=1, keepdims=True) + bb_ref[0, 0])
    bet = bet.reshape(B, MAX_VISIT, 1)
    j = lax.broadcasted_iota(F32, (1, MAX_VISIT, 1), 1)
    lam = jnp.exp(DECAY * (MAX_VISIT - j))
    attn_o[0] = jnp.sum(alpha * (bet * lam), axis=1)


def _k1(visit_node, alpha_W, alpha_b, beta_Wr, beta_b):
    nct = N_KG // _KGT
    return pl.pallas_call(
        _k1_body,
        grid=(LAYERS, nct),
        in_specs=[
            pl.BlockSpec((B, MAX_VISIT, N_KG), lambda l, c: (0, 0, 0)),
            pl.BlockSpec((1, N_KG, _KGT), lambda l, c: (l, 0, c)),
            pl.BlockSpec((1, _KGT), lambda l, c: (l, c)),
            pl.BlockSpec((1, 1, N_KG), lambda l, c: (l, 0, 0)),
            pl.BlockSpec((1, 1), lambda l, c: (l, 0)),
        ],
        out_specs=pl.BlockSpec((1, B, _KGT), lambda l, c: (l, 0, c)),
        out_shape=jax.ShapeDtypeStruct((LAYERS, B, N_KG), F32),
    )(visit_node, alpha_W, alpha_b, beta_Wr, beta_b)


def _k1b(nid_p, bat_p, xproj, attnf):
    mesh = plsc.VectorSubcoreMesh(core_axis_name="c", subcore_axis_name="s")

    @functools.partial(
        pl.kernel, mesh=mesh,
        out_type=(
            jax.ShapeDtypeStruct((NP, HID), F32),
            jax.ShapeDtypeStruct((LAYERS * NP,), F32),
        ),
        scratch_types=[
            pltpu.VMEM((64,), I32),
            pltpu.VMEM((64,), I32),
            pltpu.VMEM((64,), I32),
            pltpu.VMEM((64, HID), F32),
            pltpu.VMEM((64,), F32),
            pltpu.SemaphoreType.DMA,
        ],
    )
    def k1b(nid_hbm, bat_hbm, xproj_hbm, attnf_hbm, x0_out, an_out,
            nidv, batv, cidxv, rows, avals, sem):
        wid = lax.axis_index("s") * NC + lax.axis_index("c")
        for ci in range(NPW // 64):
            base = wid * NPW + ci * 64
            pltpu.sync_copy(nid_hbm.at[pl.ds(base, 64)], nidv)
            pltpu.sync_copy(bat_hbm.at[pl.ds(base, 64)], batv)
            pltpu.async_copy(xproj_hbm.at[nidv], rows, sem).wait()
            pltpu.sync_copy(rows, x0_out.at[pl.ds(base, 64)])
            for l in range(LAYERS):
                off = l * B * N_KG
                for q in range(4):
                    sl = pl.ds(q * 16, 16)
                    cidxv[sl] = batv[sl] * N_KG + nidv[sl] + off
                pltpu.async_copy(attnf_hbm.at[cidxv], avals, sem).wait()
                pltpu.sync_copy(avals, an_out.at[pl.ds(l * NP + base, 64)])

    return k1b(nid_p, bat_p, xproj, attnf)


def _edge_pass(xs, relmsg_l, srcp, dstp, ridp):
    mesh = plsc.VectorSubcoreMesh(core_axis_name="c", subcore_axis_name="s")
    rows_per_tile = AGG_ROWS // NS
    zrows = rows_per_tile // 2
    out_rows = N // NS

    @functools.partial(
        pl.kernel, mesh=mesh,
        out_type=jax.ShapeDtypeStruct((NC, N, HID), F32),
        scratch_types=[
            pltpu.VMEM((128,), I32),
            pltpu.VMEM((128,), I32),
            pltpu.VMEM((128,), I32),
            pltpu.VMEM((128, HID), F32),
            pltpu.VMEM((128, HID), F32),
            pltpu.VMEM((zrows, HID), F32),
            pltpu.VMEM_SHARED((AGG_ROWS, HID), F32),
            pltpu.SemaphoreType.DMA,
            pltpu.SemaphoreType.DMA,
        ],
    )
    def k2(xs_hbm, rm_hbm, src_hbm, dst_hbm, rid_hbm, out_hbm,
           srcv, dstv, ridv, xrows, mrows, zbuf, aggr, sem1, sem2):
        cid = lax.axis_index("c")
        sid = lax.axis_index("s")
        wid = sid * NC + cid

        def zb(r, carry):
            for q in range(HID // 16):
                zbuf[r, pl.ds(q * 16, 16)] = jnp.zeros((16,), F32)
            return carry

        lax.fori_loop(0, zrows, zb, 0)
        for k in range(2):
            pltpu.sync_copy(zbuf, aggr.at[pl.ds(sid * rows_per_tile + k * zrows, zrows)])
        plsc.subcore_barrier()

        def body(i, carry):
            base = wid * EPW + i * 128
            pltpu.sync_copy(src_hbm.at[pl.ds(base, 128)], srcv)
            pltpu.sync_copy(dst_hbm.at[pl.ds(base, 128)], dstv)
            pltpu.sync_copy(rid_hbm.at[pl.ds(base, 128)], ridv)
            cp1 = pltpu.async_copy(xs_hbm.at[srcv], xrows, sem1)
            cp2 = pltpu.async_copy(rm_hbm.at[ridv], mrows, sem2)
            cp1.wait()
            cp2.wait()

            def inner(j, c2):
                for q in range(HID // 16):
                    sl = pl.ds(q * 16, 16)
                    xrows[j, sl] = jnp.maximum(xrows[j, sl] + mrows[j, sl], 0.0)
                return c2

            lax.fori_loop(0, 128, inner, 0)
            pltpu.sync_copy(xrows, aggr.at[dstv], add=True)
            return carry

        lax.fori_loop(0, EPW // 128, body, 0)
        plsc.subcore_barrier()
        pltpu.sync_copy(aggr.at[pl.ds(sid * out_rows, out_rows)],
                        out_hbm.at[cid].at[pl.ds(sid * out_rows, out_rows)])

    return k2(xs, relmsg_l, srcp, dstp, ridp)


_RT = 2000


def _scale_body(x_ref, a_ref, o_ref):
    o_ref[...] = x_ref[...] * a_ref[...]


def _scale(x, a):
    return pl.pallas_call(
        _scale_body,
        grid=(N // _RT,),
        in_specs=[
            pl.BlockSpec((_RT, HID), lambda i: (i, 0)),
            pl.BlockSpec((_RT, 1), lambda i: (i, 0)),
        ],
        out_specs=pl.BlockSpec((_RT, HID), lambda i: (i, 0)),
        out_shape=jax.ShapeDtypeStruct((N, HID), F32),
    )(x, a)


def _conv_body(agg_ref, x_ref, cw_ref, cb_ref, xn_o):
    a = agg_ref[0] + agg_ref[1] + x_ref[...]
    xn_o[...] = jnp.dot(a, cw_ref[...], preferred_element_type=F32) + cb_ref[...]


def _conv_scale_body(agg_ref, x_ref, cw_ref, cb_ref, an_ref, xn_o, xs_o):
    a = agg_ref[0] + agg_ref[1] + x_ref[...]
    xn = jnp.dot(a, cw_ref[...], preferred_element_type=F32) + cb_ref[...]
    xn_o[...] = xn
    xs_o[...] = xn * an_ref[...]


def _conv(agg2, x, cw, cb):
    return pl.pallas_call(
        _conv_body,
        grid=(N // _RT,),
        in_specs=[
            pl.BlockSpec((NC, _RT, HID), lambda i: (0, i, 0)),
            pl.BlockSpec((_RT, HID), lambda i: (i, 0)),
            pl.BlockSpec((HID, HID), lambda i: (0, 0)),
            pl.BlockSpec((1, HID), lambda i: (0, 0)),
        ],
        out_specs=pl.BlockSpec((_RT, HID), lambda i: (i, 0)),
        out_shape=jax.ShapeDtypeStruct((N, HID), F32),
    )(agg2, x, cw, cb)


def _conv_scale(agg2, x, cw, cb, an):
    return pl.pallas_call(
        _conv_scale_body,
        grid=(N // _RT,),
        in_specs=[
            pl.BlockSpec((NC, _RT, HID), lambda i: (0, i, 0)),
            pl.BlockSpec((_RT, HID), lambda i: (i, 0)),
            pl.BlockSpec((HID, HID), lambda i: (0, 0)),
            pl.BlockSpec((1, HID), lambda i: (0, 0)),
            pl.BlockSpec((_RT, 1), lambda i: (i, 0)),
        ],
        out_specs=[
            pl.BlockSpec((_RT, HID), lambda i: (i, 0)),
            pl.BlockSpec((_RT, HID), lambda i: (i, 0)),
        ],
        out_shape=(
            jax.ShapeDtypeStruct((N, HID), F32),
            jax.ShapeDtypeStruct((N, HID), F32),
        ),
    )(agg2, x, cw, cb, an)


def _k4_body(x_ref, bat_ref, xnode_ref, mw_ref, mb_ref, out_ref):
    x = x_ref[...]
    oh = (bat_ref[...] == lax.broadcasted_iota(I32, (N, B), 1)).astype(F32)
    sums = lax.dot_general(oh, x, (((0,), (0,)), ((), ())),
                           preferred_element_type=F32)
    cnt = jnp.sum(oh, axis=0)
    xg = sums / jnp.maximum(cnt, 1.0)[:, None]
    cat = jnp.concatenate([xg, xnode_ref[...]], axis=1)
    out_ref[...] = jnp.sum(cat * mw_ref[...], axis=1, keepdims=True) + mb_ref[0, 0]


def _k4(x, bat2, xnode, mw, mb):
    return pl.pallas_call(
        _k4_body,
        out_shape=jax.ShapeDtypeStruct((B, 1), F32),
    )(x, bat2, xnode, mw, mb)


def kernel(node_ids, rel_ids, edge_index, batch, visit_node, ehr_nodes,
           node_emb, rel_emb, lin_W, lin_b, alpha_W, alpha_b, beta_W, beta_b,
           wr_W, wr_b, conv_W, conv_b, mlp_W, mlp_b):
    xproj, relmsg, xnode = _k0(node_emb, rel_emb, lin_W, lin_b,
                               wr_W.reshape(LAYERS, HID),
                               wr_b.reshape(LAYERS), ehr_nodes)
    attn = _k1(visit_node, alpha_W, alpha_b,
               beta_W.reshape(LAYERS, 1, N_KG), beta_b)

    nid_p = jnp.concatenate([node_ids.astype(I32), jnp.zeros((NP - N,), I32)])
    bat_p = jnp.concatenate([batch.astype(I32), jnp.zeros((NP - N,), I32)])
    x0p, anf = _k1b(nid_p, bat_p, xproj, attn.reshape(LAYERS * B * N_KG))
    x = x0p[:N]
    an = anf.reshape(LAYERS, NP)[:, :N]
    xs = _scale(x, an[0].reshape(N, 1))

    pads = EP - E
    srcp = jnp.concatenate([edge_index[0].astype(I32), jnp.zeros((pads,), I32)])
    dstp = jnp.concatenate([edge_index[1].astype(I32), jnp.full((pads,), N, I32)])
    ridp = jnp.concatenate([rel_ids.astype(I32), jnp.zeros((pads,), I32)])

    for l in range(LAYERS):
        agg2 = _edge_pass(xs, relmsg[l], srcp, dstp, ridp)
        if l < LAYERS - 1:
            x, xs = _conv_scale(agg2, x, conv_W[l], conv_b[l].reshape(1, HID),
                                an[l + 1].reshape(N, 1))
        else:
            x = _conv(agg2, x, conv_W[l], conv_b[l].reshape(1, HID))

    return _k4(x, batch.reshape(N, 1).astype(I32), xnode,
               mlp_W.reshape(1, 2 * HID), mlp_b.reshape(1, 1))


# speedup vs baseline: 6.8174x; 6.8174x over previous
import functools

import jax
import jax.numpy as jnp
from jax import lax
from jax.experimental import pallas as pl
from jax.experimental.pallas import tpu as pltpu
from jax.experimental.pallas import tpu_sc as plsc

N_KG = 2000
NUM_RELS = 100
MAX_VISIT = 20
HID = 128
LAYERS = 3
DECAY = 0.01
B = 32
N = 10000
E = 160000

NC, NS = 2, 16
NW = NC * NS
NPW = 320
NP = NW * NPW
EPW = 5120
EP = NW * EPW
AGG_ROWS = 10112
F32 = jnp.float32
I32 = jnp.int32


def _k0_body(ne_ref, re_ref, lw_ref, lb_ref, wrw_ref, wrb_ref, ehr_ref,
             xproj_o, relmsg_o, xnode_o):
    lw = lw_ref[...]
    lb = lb_ref[...][None, :]
    xproj = jnp.dot(ne_ref[...], lw, preferred_element_type=F32) + lb
    relp = jnp.dot(re_ref[...], lw, preferred_element_type=F32) + lb
    for l in range(LAYERS):
        w = jnp.sum(relp * wrw_ref[l][None, :], axis=1, keepdims=True) + wrb_ref[l]
        relmsg_o[l] = w * relp
    s = jnp.sum(ehr_ref[...], axis=1, keepdims=True)
    xn = jnp.dot(ehr_ref[...], ne_ref[...], preferred_element_type=F32) / s
    xnode_o[...] = jnp.dot(xn, lw, preferred_element_type=F32) + lb
    xproj_o[...] = xproj


def _k0(node_emb, rel_emb, lin_W, lin_b, wr_Wr, wr_br, ehr):
    return pl.pallas_call(
        _k0_body,
        out_shape=(
            jax.ShapeDtypeStruct((N_KG, HID), F32),
            jax.ShapeDtypeStruct((LAYERS, NUM_RELS, HID), F32),
            jax.ShapeDtypeStruct((B, HID), F32),
        ),
    )(node_emb, rel_emb, lin_W, lin_b, wr_Wr, wr_br, ehr)


def _k1_body(vn_ref, aw_ref, ab_ref, bw_ref, bb_ref, attn_o):
    vn2 = vn_ref[...].reshape(B * MAX_VISIT, N_KG)
    logits = jnp.dot(vn2, aw_ref[0], preferred_element_type=F32) + ab_ref[0]
    logits = logits.reshape(B, MAX_VISIT, N_KG)
    m = jnp.max(logits, axis=1, keepdims=True)
    ex = jnp.exp(logits - m)
    alpha = ex / jnp.sum(ex, axis=1, keepdims=True)
    bet = jnp.tanh(jnp.sum(vn2 * bw_ref[0], axis=1, keepdims=True) + bb_ref[0, 0, 0])
    bet = bet.reshape(B, MAX_VISIT, 1)
    j = lax.broadcasted_iota(I32, (1, MAX_VISIT, 1), 1).astype(F32)
    lam = jnp.exp(DECAY * (MAX_VISIT - j))
    attn_o[0] = jnp.sum(alpha * (bet * lam), axis=1)


def _k1(visit_node, alpha_W, alpha_b3, beta_Wr, beta_b3):
    return pl.pallas_call(
        _k1_body,
        grid=(LAYERS,),
        in_specs=[
            pl.BlockSpec((B, MAX_VISIT, N_KG), lambda l: (0, 0, 0)),
            pl.BlockSpec((1, N_KG, N_KG), lambda l: (l, 0, 0)),
            pl.BlockSpec((1, 1, N_KG), lambda l: (l, 0, 0)),
            pl.BlockSpec((1, 1, N_KG), lambda l: (l, 0, 0)),
            pl.BlockSpec((1, 1, 1), lambda l: (l, 0, 0)),
        ],
        out_specs=pl.BlockSpec((1, B, N_KG), lambda l: (l, 0, 0)),
        out_shape=jax.ShapeDtypeStruct((LAYERS, B, N_KG), F32),
    )(visit_node, alpha_W, alpha_b3, beta_Wr, beta_b3)


def _k1b(nid_p, bat_p, xproj, attnf):
    mesh = plsc.VectorSubcoreMesh(core_axis_name="c", subcore_axis_name="s")

    @functools.partial(
        pl.kernel, mesh=mesh,
        out_type=(
            jax.ShapeDtypeStruct((NP, HID), F32),
            jax.ShapeDtypeStruct((LAYERS * NP,), F32),
        ),
        scratch_types=[
            pltpu.VMEM((64,), I32),
            pltpu.VMEM((64,), I32),
            pltpu.VMEM((64,), I32),
            pltpu.VMEM((64, HID), F32),
            pltpu.VMEM((64,), F32),
            pltpu.SemaphoreType.DMA,
        ],
    )
    def k1b(nid_hbm, bat_hbm, xproj_hbm, attnf_hbm, x0_out, an_out,
            nidv, batv, cidxv, rows, avals, sem):
        wid = lax.axis_index("s") * NC + lax.axis_index("c")
        for ci in range(NPW // 64):
            base = wid * NPW + ci * 64
            pltpu.sync_copy(nid_hbm.at[pl.ds(base, 64)], nidv)
            pltpu.sync_copy(bat_hbm.at[pl.ds(base, 64)], batv)
            pltpu.async_copy(xproj_hbm.at[nidv], rows, sem).wait()
            pltpu.sync_copy(rows, x0_out.at[pl.ds(base, 64)])
            for l in range(LAYERS):
                off = l * B * N_KG
                for q in range(4):
                    sl = pl.ds(q * 16, 16)
                    cidxv[sl] = batv[sl] * N_KG + nidv[sl] + off
                pltpu.async_copy(attnf_hbm.at[cidxv], avals, sem).wait()
                pltpu.sync_copy(avals, an_out.at[pl.ds(l * NP + base, 64)])

    return k1b(nid_p, bat_p, xproj, attnf)


def _edge_pass(xs, relmsg_l, srcp, dstp, ridp):
    mesh = plsc.VectorSubcoreMesh(core_axis_name="c", subcore_axis_name="s")
    rows_per_tile = AGG_ROWS // NS

    @functools.partial(
        pl.kernel, mesh=mesh,
        out_type=jax.ShapeDtypeStruct((NC, AGG_ROWS, HID), F32),
        scratch_types=[
            pltpu.VMEM((128,), I32),
            pltpu.VMEM((128,), I32),
            pltpu.VMEM((128,), I32),
            pltpu.VMEM((128, HID), F32),
            pltpu.VMEM((128, HID), F32),
            pltpu.VMEM_SHARED((AGG_ROWS, HID), F32),
            pltpu.SemaphoreType.DMA,
            pltpu.SemaphoreType.DMA,
        ],
    )
    def k2(xs_hbm, rm_hbm, src_hbm, dst_hbm, rid_hbm, out_hbm,
           srcv, dstv, ridv, xrows, mrows, aggr, sem1, sem2):
        cid = lax.axis_index("c")
        sid = lax.axis_index("s")
        wid = sid * NC + cid

        def zb(r, carry):
            for q in range(HID // 16):
                xrows[r, pl.ds(q * 16, 16)] = jnp.zeros((16,), F32)
            return carry

        lax.fori_loop(0, 128, zb, 0)
        for k in range(4):
            pltpu.sync_copy(xrows, aggr.at[pl.ds(sid * rows_per_tile + k * 128, 128)])
        pltpu.sync_copy(xrows.at[pl.ds(0, rows_per_tile - 512)],
                        aggr.at[pl.ds(sid * rows_per_tile + 512, rows_per_tile - 512)])
        plsc.subcore_barrier()

        def body(i, carry):
            base = wid * EPW + i * 128
            pltpu.sync_copy(src_hbm.at[pl.ds(base, 128)], srcv)
            pltpu.sync_copy(dst_hbm.at[pl.ds(base, 128)], dstv)
            pltpu.sync_copy(rid_hbm.at[pl.ds(base, 128)], ridv)
            cp1 = pltpu.async_copy(xs_hbm.at[srcv], xrows, sem1)
            cp2 = pltpu.async_copy(rm_hbm.at[ridv], mrows, sem2)
            cp1.wait()
            cp2.wait()

            def inner(j, c2):
                for q in range(HID // 16):
                    sl = pl.ds(q * 16, 16)
                    xrows[j, sl] = jnp.maximum(xrows[j, sl] + mrows[j, sl], 0.0)
                return c2

            lax.fori_loop(0, 128, inner, 0)
            pltpu.sync_copy(xrows, aggr.at[dstv], add=True)
            return carry

        lax.fori_loop(0, EPW // 128, body, 0)
        plsc.subcore_barrier()
        pltpu.sync_copy(aggr.at[pl.ds(sid * rows_per_tile, rows_per_tile)],
                        out_hbm.at[cid].at[pl.ds(sid * rows_per_tile, rows_per_tile)])

    return k2(xs, relmsg_l, srcp, dstp, ridp)


_RT = 2000


def _scale_body(x_ref, a_ref, o_ref):
    o_ref[...] = x_ref[...] * a_ref[...]


def _scale(x, a):
    return pl.pallas_call(
        _scale_body,
        grid=(N // _RT,),
        in_specs=[
            pl.BlockSpec((_RT, HID), lambda i: (i, 0)),
            pl.BlockSpec((_RT, 1), lambda i: (i, 0)),
        ],
        out_specs=pl.BlockSpec((_RT, HID), lambda i: (i, 0)),
        out_shape=jax.ShapeDtypeStruct((N, HID), F32),
    )(x, a)


def _conv_body(agg_ref, x_ref, cw_ref, cb_ref, xn_o):
    a = agg_ref[0] + agg_ref[1] + x_ref[...]
    xn_o[...] = jnp.dot(a, cw_ref[...], preferred_element_type=F32) + cb_ref[...]


def _conv_scale_body(agg_ref, x_ref, cw_ref, cb_ref, an_ref, xn_o, xs_o):
    a = agg_ref[0] + agg_ref[1] + x_ref[...]
    xn = jnp.dot(a, cw_ref[...], preferred_element_type=F32) + cb_ref[...]
    xn_o[...] = xn
    xs_o[...] = xn * an_ref[...]


def _conv(agg2, x, cw, cb):
    return pl.pallas_call(
        _conv_body,
        grid=(N // _RT,),
        in_specs=[
            pl.BlockSpec((NC, _RT, HID), lambda i: (0, i, 0)),
            pl.BlockSpec((_RT, HID), lambda i: (i, 0)),
            pl.BlockSpec((HID, HID), lambda i: (0, 0)),
            pl.BlockSpec((1, HID), lambda i: (0, 0)),
        ],
        out_specs=pl.BlockSpec((_RT, HID), lambda i: (i, 0)),
        out_shape=jax.ShapeDtypeStruct((N, HID), F32),
    )(agg2, x, cw, cb)


def _conv_scale(agg2, x, cw, cb, an):
    return pl.pallas_call(
        _conv_scale_body,
        grid=(N // _RT,),
        in_specs=[
            pl.BlockSpec((NC, _RT, HID), lambda i: (0, i, 0)),
            pl.BlockSpec((_RT, HID), lambda i: (i, 0)),
            pl.BlockSpec((HID, HID), lambda i: (0, 0)),
            pl.BlockSpec((1, HID), lambda i: (0, 0)),
            pl.BlockSpec((_RT, 1), lambda i: (i, 0)),
        ],
        out_specs=[
            pl.BlockSpec((_RT, HID), lambda i: (i, 0)),
            pl.BlockSpec((_RT, HID), lambda i: (i, 0)),
        ],
        out_shape=(
            jax.ShapeDtypeStruct((N, HID), F32),
            jax.ShapeDtypeStruct((N, HID), F32),
        ),
    )(agg2, x, cw, cb, an)


def _k4_body(x_ref, bat_ref, xnode_ref, mw_ref, mb_ref, out_ref):
    x = x_ref[...]
    oh = (bat_ref[...] == lax.broadcasted_iota(I32, (N, B), 1)).astype(F32)
    sums = lax.dot_general(oh, x, (((0,), (0,)), ((), ())),
                           preferred_element_type=F32)
    cnt = jnp.sum(oh, axis=0)
    xg = sums / jnp.maximum(cnt, 1.0)[:, None]
    cat = jnp.concatenate([xg, xnode_ref[...]], axis=1)
    out_ref[...] = jnp.sum(cat * mw_ref[...], axis=1, keepdims=True) + mb_ref[0, 0]


def _k4(x, bat2, xnode, mw, mb):
    return pl.pallas_call(
        _k4_body,
        out_shape=jax.ShapeDtypeStruct((B, 1), F32),
    )(x, bat2, xnode, mw, mb)


def kernel(node_ids, rel_ids, edge_index, batch, visit_node, ehr_nodes,
           node_emb, rel_emb, lin_W, lin_b, alpha_W, alpha_b, beta_W, beta_b,
           wr_W, wr_b, conv_W, conv_b, mlp_W, mlp_b):
    xproj, relmsg, xnode = _k0(node_emb, rel_emb, lin_W, lin_b,
                               wr_W.reshape(LAYERS, HID),
                               wr_b.reshape(LAYERS), ehr_nodes)
    attn = _k1(visit_node, alpha_W, alpha_b.reshape(LAYERS, 1, N_KG),
               beta_W.reshape(LAYERS, 1, N_KG), beta_b.reshape(LAYERS, 1, 1))

    nid_p = jnp.concatenate([node_ids.astype(I32), jnp.zeros((NP - N,), I32)])
    bat_p = jnp.concatenate([batch.astype(I32), jnp.zeros((NP - N,), I32)])
    x0p, anf = _k1b(nid_p, bat_p, xproj, attn.reshape(LAYERS * B * N_KG))
    x = x0p[:N]
    an = anf.reshape(LAYERS, NP)[:, :N]
    xs = _scale(x, an[0].reshape(N, 1))

    pads = EP - E
    srcp = jnp.concatenate([edge_index[0].astype(I32), jnp.zeros((pads,), I32)])
    dstp = jnp.concatenate([edge_index[1].astype(I32), jnp.full((pads,), N, I32)])
    ridp = jnp.concatenate([rel_ids.astype(I32), jnp.zeros((pads,), I32)])

    for l in range(LAYERS):
        agg2 = _edge_pass(xs, relmsg[l], srcp, dstp, ridp)[:, :N]
        if l < LAYERS - 1:
            x, xs = _conv_scale(agg2, x, conv_W[l], conv_b[l].reshape(1, HID),
                                an[l + 1].reshape(N, 1))
        else:
            x = _conv(agg2, x, conv_W[l], conv_b[l].reshape(1, HID))

    return _k4(x, batch.reshape(N, 1).astype(I32), xnode,
               mlp_W.reshape(1, 2 * HID), mlp_b.reshape(1, 1))


# speedup vs baseline: 7.7807x; 1.1413x over previous
import functools

import jax
import jax.numpy as jnp
from jax import lax
from jax.experimental import pallas as pl
from jax.experimental.pallas import tpu as pltpu
from jax.experimental.pallas import tpu_sc as plsc

N_KG = 2000
NUM_RELS = 100
MAX_VISIT = 20
HID = 128
LAYERS = 3
DECAY = 0.01
B = 32
N = 10000
E = 160000

NC, NS = 2, 16
NW = NC * NS
NPW = 320
NP = NW * NPW
EPW = 5120
EP = NW * EPW
AGG_ROWS = 10112
F32 = jnp.float32
I32 = jnp.int32


def _k0_body(ne_ref, re_ref, lw_ref, lb_ref, wrw_ref, wrb_ref, ehr_ref,
             xproj_o, relmsg_o, xnode_o):
    lw = lw_ref[...]
    lb = lb_ref[...][None, :]
    xproj = jnp.dot(ne_ref[...], lw, preferred_element_type=F32) + lb
    relp = jnp.dot(re_ref[...], lw, preferred_element_type=F32) + lb
    for l in range(LAYERS):
        w = jnp.sum(relp * wrw_ref[l][None, :], axis=1, keepdims=True) + wrb_ref[l]
        relmsg_o[l] = w * relp
    s = jnp.sum(ehr_ref[...], axis=1, keepdims=True)
    xn = jnp.dot(ehr_ref[...], ne_ref[...], preferred_element_type=F32) / s
    xnode_o[...] = jnp.dot(xn, lw, preferred_element_type=F32) + lb
    xproj_o[...] = xproj


def _k0(node_emb, rel_emb, lin_W, lin_b, wr_Wr, wr_br, ehr):
    return pl.pallas_call(
        _k0_body,
        out_shape=(
            jax.ShapeDtypeStruct((N_KG, HID), F32),
            jax.ShapeDtypeStruct((LAYERS, NUM_RELS, HID), F32),
            jax.ShapeDtypeStruct((B, HID), F32),
        ),
    )(node_emb, rel_emb, lin_W, lin_b, wr_Wr, wr_br, ehr)


def _k1_body(vn_ref, aw_ref, ab_ref, bw_ref, bb_ref, attn_o):
    vn2 = vn_ref[...].reshape(B * MAX_VISIT, N_KG)
    logits = jnp.dot(vn2, aw_ref[0], preferred_element_type=F32) + ab_ref[0]
    logits = logits.reshape(B, MAX_VISIT, N_KG)
    m = jnp.max(logits, axis=1, keepdims=True)
    ex = jnp.exp(logits - m)
    alpha = ex / jnp.sum(ex, axis=1, keepdims=True)
    bet = jnp.tanh(jnp.sum(vn2 * bw_ref[0], axis=1, keepdims=True) + bb_ref[0, 0, 0])
    bet = bet.reshape(B, MAX_VISIT, 1)
    j = lax.broadcasted_iota(I32, (1, MAX_VISIT, 1), 1).astype(F32)
    lam = jnp.exp(DECAY * (MAX_VISIT - j))
    attn_o[0] = jnp.sum(alpha * (bet * lam), axis=1)


def _k1(visit_node, alpha_W, alpha_b3, beta_Wr, beta_b3):
    return pl.pallas_call(
        _k1_body,
        grid=(LAYERS,),
        in_specs=[
            pl.BlockSpec((B, MAX_VISIT, N_KG), lambda l: (0, 0, 0)),
            pl.BlockSpec((1, N_KG, N_KG), lambda l: (l, 0, 0)),
            pl.BlockSpec((1, 1, N_KG), lambda l: (l, 0, 0)),
            pl.BlockSpec((1, 1, N_KG), lambda l: (l, 0, 0)),
            pl.BlockSpec((1, 1, 1), lambda l: (l, 0, 0)),
        ],
        out_specs=pl.BlockSpec((1, B, N_KG), lambda l: (l, 0, 0)),
        out_shape=jax.ShapeDtypeStruct((LAYERS, B, N_KG), F32),
    )(visit_node, alpha_W, alpha_b3, beta_Wr, beta_b3)


def _k1b(nid_p, bat_p, xproj, attnf):
    mesh = plsc.VectorSubcoreMesh(core_axis_name="c", subcore_axis_name="s")

    @functools.partial(
        pl.kernel, mesh=mesh,
        out_type=(
            jax.ShapeDtypeStruct((NP, HID), F32),
            jax.ShapeDtypeStruct((LAYERS * NP,), F32),
        ),
        scratch_types=[
            pltpu.VMEM((64,), I32),
            pltpu.VMEM((64,), I32),
            pltpu.VMEM((64,), I32),
            pltpu.VMEM((64, HID), F32),
            pltpu.VMEM((64,), F32),
            pltpu.SemaphoreType.DMA,
        ],
    )
    def k1b(nid_hbm, bat_hbm, xproj_hbm, attnf_hbm, x0_out, an_out,
            nidv, batv, cidxv, rows, avals, sem):
        wid = lax.axis_index("s") * NC + lax.axis_index("c")
        for ci in range(NPW // 64):
            base = wid * NPW + ci * 64
            pltpu.sync_copy(nid_hbm.at[pl.ds(base, 64)], nidv)
            pltpu.sync_copy(bat_hbm.at[pl.ds(base, 64)], batv)
            pltpu.async_copy(xproj_hbm.at[nidv], rows, sem).wait()
            pltpu.sync_copy(rows, x0_out.at[pl.ds(base, 64)])
            for l in range(LAYERS):
                off = l * B * N_KG
                for q in range(4):
                    sl = pl.ds(q * 16, 16)
                    cidxv[sl] = batv[sl] * N_KG + nidv[sl] + off
                pltpu.async_copy(attnf_hbm.at[cidxv], avals, sem).wait()
                pltpu.sync_copy(avals, an_out.at[pl.ds(l * NP + base, 64)])

    return k1b(nid_p, bat_p, xproj, attnf)


CH = 80
NCH = EPW // CH


def _edge_pass(xs, relmsg_l, idx3):
    mesh = plsc.VectorSubcoreMesh(core_axis_name="c", subcore_axis_name="s")
    rows_per_tile = AGG_ROWS // NS

    @functools.partial(
        pl.kernel, mesh=mesh,
        out_type=jax.ShapeDtypeStruct((NC, AGG_ROWS, HID), F32),
        scratch_types=[
            pltpu.VMEM((3, CH), I32),
            pltpu.VMEM((3, CH), I32),
            pltpu.VMEM((CH, HID), F32),
            pltpu.VMEM((CH, HID), F32),
            pltpu.VMEM((CH, HID), F32),
            pltpu.VMEM((CH, HID), F32),
            pltpu.VMEM_SHARED((AGG_ROWS, HID), F32),
            pltpu.SemaphoreType.DMA,
            pltpu.SemaphoreType.DMA,
            pltpu.SemaphoreType.DMA,
            pltpu.SemaphoreType.DMA,
        ],
    )
    def k2(xs_hbm, rm_hbm, idx3_hbm, out_hbm,
           idxa, idxb, xra, xrb, mra, mrb, aggr, sxa, sxb, sma, smb):
        cid = lax.axis_index("c")
        sid = lax.axis_index("s")
        wid = sid * NC + cid
        cbase = wid * NCH

        def zb(r, carry):
            for q in range(HID // 16):
                xra[r, pl.ds(q * 16, 16)] = jnp.zeros((16,), F32)
            return carry

        lax.fori_loop(0, CH, zb, 0)
        for k in range(rows_per_tile // CH):
            pltpu.sync_copy(xra, aggr.at[pl.ds(sid * rows_per_tile + k * CH, CH)])
        rem = rows_per_tile % CH
        pltpu.sync_copy(xra.at[pl.ds(0, rem)],
                        aggr.at[pl.ds(sid * rows_per_tile + rows_per_tile - rem, rem)])
        plsc.subcore_barrier()

        def start(idx, xr, mr, sx, sm):
            pltpu.async_copy(xs_hbm.at[idx.at[0]], xr, sx)
            pltpu.async_copy(rm_hbm.at[idx.at[2]], mr, sm)

        def wait(idx, xr, mr, sx, sm):
            pltpu.make_async_copy(xs_hbm.at[idx.at[0]], xr, sx).wait()
            pltpu.make_async_copy(rm_hbm.at[idx.at[2]], mr, sm).wait()

        def crunch(idx, xr, mr):
            def inner(j, c2):
                for q in range(HID // 16):
                    sl = pl.ds(q * 16, 16)
                    xr[j, sl] = jnp.maximum(xr[j, sl] + mr[j, sl], 0.0)
                return c2

            lax.fori_loop(0, CH, inner, 0)
            pltpu.sync_copy(xr, aggr.at[idx.at[1]], add=True)

        pltpu.sync_copy(idx3_hbm.at[cbase], idxa)
        start(idxa, xra, mra, sxa, sma)
        pltpu.sync_copy(idx3_hbm.at[cbase + 1], idxb)

        def pair(p, carry):
            c0 = 2 * p
            start(idxb, xrb, mrb, sxb, smb)
            wait(idxa, xra, mra, sxa, sma)
            crunch(idxa, xra, mra)

            @pl.when(c0 + 2 < NCH)
            def _():
                pltpu.sync_copy(idx3_hbm.at[cbase + c0 + 2], idxa)
                start(idxa, xra, mra, sxa, sma)

            wait(idxb, xrb, mrb, sxb, smb)
            crunch(idxb, xrb, mrb)

            @pl.when(c0 + 3 < NCH)
            def _():
                pltpu.sync_copy(idx3_hbm.at[cbase + c0 + 3], idxb)

            return carry

        lax.fori_loop(0, NCH // 2, pair, 0)
        plsc.subcore_barrier()
        pltpu.sync_copy(aggr.at[pl.ds(sid * rows_per_tile, rows_per_tile)],
                        out_hbm.at[cid].at[pl.ds(sid * rows_per_tile, rows_per_tile)])

    return k2(xs, relmsg_l, idx3)


_RT = 2000


def _scale_body(x_ref, a_ref, o_ref):
    o_ref[...] = x_ref[...] * a_ref[...]


def _scale(x, a):
    return pl.pallas_call(
        _scale_body,
        grid=(N // _RT,),
        in_specs=[
            pl.BlockSpec((_RT, HID), lambda i: (i, 0)),
            pl.BlockSpec((_RT, 1), lambda i: (i, 0)),
        ],
        out_specs=pl.BlockSpec((_RT, HID), lambda i: (i, 0)),
        out_shape=jax.ShapeDtypeStruct((N, HID), F32),
    )(x, a)


def _conv_body(agg_ref, x_ref, cw_ref, cb_ref, xn_o):
    a = agg_ref[0] + agg_ref[1] + x_ref[...]
    xn_o[...] = jnp.dot(a, cw_ref[...], preferred_element_type=F32) + cb_ref[...]


def _conv_scale_body(agg_ref, x_ref, cw_ref, cb_ref, an_ref, xn_o, xs_o):
    a = agg_ref[0] + agg_ref[1] + x_ref[...]
    xn = jnp.dot(a, cw_ref[...], preferred_element_type=F32) + cb_ref[...]
    xn_o[...] = xn
    xs_o[...] = xn * an_ref[...]


def _conv(agg2, x, cw, cb):
    return pl.pallas_call(
        _conv_body,
        grid=(N // _RT,),
        in_specs=[
            pl.BlockSpec((NC, _RT, HID), lambda i: (0, i, 0)),
            pl.BlockSpec((_RT, HID), lambda i: (i, 0)),
            pl.BlockSpec((HID, HID), lambda i: (0, 0)),
            pl.BlockSpec((1, HID), lambda i: (0, 0)),
        ],
        out_specs=pl.BlockSpec((_RT, HID), lambda i: (i, 0)),
        out_shape=jax.ShapeDtypeStruct((N, HID), F32),
    )(agg2, x, cw, cb)


def _conv_scale(agg2, x, cw, cb, an):
    return pl.pallas_call(
        _conv_scale_body,
        grid=(N // _RT,),
        in_specs=[
            pl.BlockSpec((NC, _RT, HID), lambda i: (0, i, 0)),
            pl.BlockSpec((_RT, HID), lambda i: (i, 0)),
            pl.BlockSpec((HID, HID), lambda i: (0, 0)),
            pl.BlockSpec((1, HID), lambda i: (0, 0)),
            pl.BlockSpec((_RT, 1), lambda i: (i, 0)),
        ],
        out_specs=[
            pl.BlockSpec((_RT, HID), lambda i: (i, 0)),
            pl.BlockSpec((_RT, HID), lambda i: (i, 0)),
        ],
        out_shape=(
            jax.ShapeDtypeStruct((N, HID), F32),
            jax.ShapeDtypeStruct((N, HID), F32),
        ),
    )(agg2, x, cw, cb, an)


def _k4_body(x_ref, bat_ref, xnode_ref, mw_ref, mb_ref, out_ref):
    x = x_ref[...]
    oh = (bat_ref[...] == lax.broadcasted_iota(I32, (N, B), 1)).astype(F32)
    sums = lax.dot_general(oh, x, (((0,), (0,)), ((), ())),
                           preferred_element_type=F32)
    cnt = jnp.sum(oh, axis=0)
    xg = sums / jnp.maximum(cnt, 1.0)[:, None]
    cat = jnp.concatenate([xg, xnode_ref[...]], axis=1)
    out_ref[...] = jnp.sum(cat * mw_ref[...], axis=1, keepdims=True) + mb_ref[0, 0]


def _k4(x, bat2, xnode, mw, mb):
    return pl.pallas_call(
        _k4_body,
        out_shape=jax.ShapeDtypeStruct((B, 1), F32),
    )(x, bat2, xnode, mw, mb)


def kernel(node_ids, rel_ids, edge_index, batch, visit_node, ehr_nodes,
           node_emb, rel_emb, lin_W, lin_b, alpha_W, alpha_b, beta_W, beta_b,
           wr_W, wr_b, conv_W, conv_b, mlp_W, mlp_b):
    xproj, relmsg, xnode = _k0(node_emb, rel_emb, lin_W, lin_b,
                               wr_W.reshape(LAYERS, HID),
                               wr_b.reshape(LAYERS), ehr_nodes)
    attn = _k1(visit_node, alpha_W, alpha_b.reshape(LAYERS, 1, N_KG),
               beta_W.reshape(LAYERS, 1, N_KG), beta_b.reshape(LAYERS, 1, 1))

    nid_p = jnp.concatenate([node_ids.astype(I32), jnp.zeros((NP - N,), I32)])
    bat_p = jnp.concatenate([batch.astype(I32), jnp.zeros((NP - N,), I32)])
    x0p, anf = _k1b(nid_p, bat_p, xproj, attn.reshape(LAYERS * B * N_KG))
    x = x0p[:N]
    an = anf.reshape(LAYERS, NP)[:, :N]
    xs = _scale(x, an[0].reshape(N, 1))

    pads = EP - E
    srcp = jnp.concatenate([edge_index[0].astype(I32), jnp.zeros((pads,), I32)])
    dstp = jnp.concatenate([edge_index[1].astype(I32), jnp.full((pads,), N, I32)])
    ridp = jnp.concatenate([rel_ids.astype(I32), jnp.zeros((pads,), I32)])
    idx3 = jnp.stack([srcp.reshape(-1, CH), dstp.reshape(-1, CH),
                      ridp.reshape(-1, CH)], axis=1)

    for l in range(LAYERS):
        agg2 = _edge_pass(xs, relmsg[l], idx3)[:, :N]
        if l < LAYERS - 1:
            x, xs = _conv_scale(agg2, x, conv_W[l], conv_b[l].reshape(1, HID),
                                an[l + 1].reshape(N, 1))
        else:
            x = _conv(agg2, x, conv_W[l], conv_b[l].reshape(1, HID))

    return _k4(x, batch.reshape(N, 1).astype(I32), xnode,
               mlp_W.reshape(1, 2 * HID), mlp_b.reshape(1, 1))


# speedup vs baseline: 8.5611x; 1.1003x over previous
import functools

import jax
import jax.numpy as jnp
from jax import lax
from jax.experimental import pallas as pl
from jax.experimental.pallas import tpu as pltpu
from jax.experimental.pallas import tpu_sc as plsc

N_KG = 2000
NUM_RELS = 100
MAX_VISIT = 20
HID = 128
LAYERS = 3
DECAY = 0.01
B = 32
N = 10000
E = 160000

NC, NS = 2, 16
NW = NC * NS
NPW = 320
NP = NW * NPW
EPW = 5120
EP = NW * EPW
AGG_ROWS = 10112
F32 = jnp.float32
I32 = jnp.int32


def _k0_body(ne_ref, re_ref, lw_ref, lb_ref, wrw_ref, wrb_ref, ehr_ref,
             xproj_o, relmsg_o, xnode_o):
    lw = lw_ref[...]
    lb = lb_ref[...][None, :]
    xproj = jnp.dot(ne_ref[...], lw, preferred_element_type=F32) + lb
    relp = jnp.dot(re_ref[...], lw, preferred_element_type=F32) + lb
    for l in range(LAYERS):
        w = jnp.sum(relp * wrw_ref[l][None, :], axis=1, keepdims=True) + wrb_ref[l]
        relmsg_o[l] = w * relp
    s = jnp.sum(ehr_ref[...], axis=1, keepdims=True)
    xn = jnp.dot(ehr_ref[...], ne_ref[...], preferred_element_type=F32) / s
    xnode_o[...] = jnp.dot(xn, lw, preferred_element_type=F32) + lb
    xproj_o[...] = xproj


def _k0(node_emb, rel_emb, lin_W, lin_b, wr_Wr, wr_br, ehr):
    return pl.pallas_call(
        _k0_body,
        out_shape=(
            jax.ShapeDtypeStruct((N_KG, HID), F32),
            jax.ShapeDtypeStruct((LAYERS, NUM_RELS, HID), F32),
            jax.ShapeDtypeStruct((B, HID), F32),
        ),
    )(node_emb, rel_emb, lin_W, lin_b, wr_Wr, wr_br, ehr)


def _k1_body(vn_ref, aw_ref, ab_ref, bw_ref, bb_ref, attn_o):
    vn2 = vn_ref[...].reshape(B * MAX_VISIT, N_KG)
    logits = jnp.dot(vn2, aw_ref[0], preferred_element_type=F32) + ab_ref[0]
    logits = logits.reshape(B, MAX_VISIT, N_KG)
    m = jnp.max(logits, axis=1, keepdims=True)
    ex = jnp.exp(logits - m)
    alpha = ex / jnp.sum(ex, axis=1, keepdims=True)
    bet = jnp.tanh(jnp.sum(vn2 * bw_ref[0], axis=1, keepdims=True) + bb_ref[0, 0, 0])
    bet = bet.reshape(B, MAX_VISIT, 1)
    j = lax.broadcasted_iota(I32, (1, MAX_VISIT, 1), 1).astype(F32)
    lam = jnp.exp(DECAY * (MAX_VISIT - j))
    attn_o[0] = jnp.sum(alpha * (bet * lam), axis=1)


def _k1(visit_node, alpha_W, alpha_b3, beta_Wr, beta_b3):
    return pl.pallas_call(
        _k1_body,
        grid=(LAYERS,),
        in_specs=[
            pl.BlockSpec((B, MAX_VISIT, N_KG), lambda l: (0, 0, 0)),
            pl.BlockSpec((1, N_KG, N_KG), lambda l: (l, 0, 0)),
            pl.BlockSpec((1, 1, N_KG), lambda l: (l, 0, 0)),
            pl.BlockSpec((1, 1, N_KG), lambda l: (l, 0, 0)),
            pl.BlockSpec((1, 1, 1), lambda l: (l, 0, 0)),
        ],
        out_specs=pl.BlockSpec((1, B, N_KG), lambda l: (l, 0, 0)),
        out_shape=jax.ShapeDtypeStruct((LAYERS, B, N_KG), F32),
    )(visit_node, alpha_W, alpha_b3, beta_Wr, beta_b3)


def _k1b(nid_p, bat_p, xproj, attnf):
    mesh = plsc.VectorSubcoreMesh(core_axis_name="c", subcore_axis_name="s")

    @functools.partial(
        pl.kernel, mesh=mesh,
        out_type=(
            jax.ShapeDtypeStruct((NP, HID), F32),
            jax.ShapeDtypeStruct((LAYERS * NP,), F32),
        ),
        scratch_types=[
            pltpu.VMEM((64,), I32),
            pltpu.VMEM((64,), I32),
            pltpu.VMEM((64,), I32),
            pltpu.VMEM((64, HID), F32),
            pltpu.VMEM((64,), F32),
            pltpu.SemaphoreType.DMA,
        ],
    )
    def k1b(nid_hbm, bat_hbm, xproj_hbm, attnf_hbm, x0_out, an_out,
            nidv, batv, cidxv, rows, avals, sem):
        wid = lax.axis_index("s") * NC + lax.axis_index("c")
        for ci in range(NPW // 64):
            base = wid * NPW + ci * 64
            pltpu.sync_copy(nid_hbm.at[pl.ds(base, 64)], nidv)
            pltpu.sync_copy(bat_hbm.at[pl.ds(base, 64)], batv)
            pltpu.async_copy(xproj_hbm.at[nidv], rows, sem).wait()
            pltpu.sync_copy(rows, x0_out.at[pl.ds(base, 64)])
            for l in range(LAYERS):
                off = l * B * N_KG
                for q in range(4):
                    sl = pl.ds(q * 16, 16)
                    cidxv[sl] = batv[sl] * N_KG + nidv[sl] + off
                pltpu.async_copy(attnf_hbm.at[cidxv], avals, sem).wait()
                pltpu.sync_copy(avals, an_out.at[pl.ds(l * NP + base, 64)])

    return k1b(nid_p, bat_p, xproj, attnf)


CH = 80
NCH = EPW // CH


def _edge_pass(xs, relmsg_l, idx3):
    mesh = plsc.VectorSubcoreMesh(core_axis_name="c", subcore_axis_name="s")
    rows_per_tile = AGG_ROWS // NS

    @functools.partial(
        pl.kernel, mesh=mesh,
        out_type=jax.ShapeDtypeStruct((NC, AGG_ROWS, HID), F32),
        scratch_types=[
            pltpu.VMEM((3, CH), I32),
            pltpu.VMEM((3, CH), I32),
            pltpu.VMEM((CH, HID), F32),
            pltpu.VMEM((CH, HID), F32),
            pltpu.VMEM((CH, HID), F32),
            pltpu.VMEM((CH, HID), F32),
            pltpu.VMEM_SHARED((AGG_ROWS, HID), F32),
            pltpu.VMEM_SHARED((NUM_RELS, HID), F32),
            pltpu.SemaphoreType.DMA,
            pltpu.SemaphoreType.DMA,
            pltpu.SemaphoreType.DMA,
            pltpu.SemaphoreType.DMA,
        ],
    )
    def k2(xs_hbm, rm_hbm, idx3_hbm, out_hbm,
           idxa, idxb, xra, xrb, mra, mrb, aggr, rmsh, sxa, sxb, sma, smb):
        cid = lax.axis_index("c")
        sid = lax.axis_index("s")
        wid = sid * NC + cid
        cbase = wid * NCH

        def zb(r, carry):
            for q in range(HID // 16):
                xra[r, pl.ds(q * 16, 16)] = jnp.zeros((16,), F32)
            return carry

        @pl.when(sid == 0)
        def _():
            pltpu.sync_copy(rm_hbm, rmsh)

        lax.fori_loop(0, CH, zb, 0)
        for k in range(rows_per_tile // CH):
            pltpu.sync_copy(xra, aggr.at[pl.ds(sid * rows_per_tile + k * CH, CH)])
        rem = rows_per_tile % CH
        pltpu.sync_copy(xra.at[pl.ds(0, rem)],
                        aggr.at[pl.ds(sid * rows_per_tile + rows_per_tile - rem, rem)])
        plsc.subcore_barrier()

        def start(idx, xr, mr, sx, sm):
            pltpu.async_copy(xs_hbm.at[idx.at[0]], xr, sx)
            pltpu.async_copy(rmsh.at[idx.at[2]], mr, sm)

        def wait(idx, xr, mr, sx, sm):
            pltpu.make_async_copy(xs_hbm.at[idx.at[0]], xr, sx).wait()
            pltpu.make_async_copy(rmsh.at[idx.at[2]], mr, sm).wait()

        def crunch(idx, xr, mr):
            def inner(j, c2):
                for q in range(HID // 16):
                    sl = pl.ds(q * 16, 16)
                    xr[j, sl] = jnp.maximum(xr[j, sl] + mr[j, sl], 0.0)
                return c2

            lax.fori_loop(0, CH, inner, 0)
            pltpu.sync_copy(xr, aggr.at[idx.at[1]], add=True)

        pltpu.sync_copy(idx3_hbm.at[cbase], idxa)
        start(idxa, xra, mra, sxa, sma)
        pltpu.sync_copy(idx3_hbm.at[cbase + 1], idxb)

        def pair(p, carry):
            c0 = 2 * p
            start(idxb, xrb, mrb, sxb, smb)
            wait(idxa, xra, mra, sxa, sma)
            crunch(idxa, xra, mra)

            @pl.when(c0 + 2 < NCH)
            def _():
                pltpu.sync_copy(idx3_hbm.at[cbase + c0 + 2], idxa)
                start(idxa, xra, mra, sxa, sma)

            wait(idxb, xrb, mrb, sxb, smb)
            crunch(idxb, xrb, mrb)

            @pl.when(c0 + 3 < NCH)
            def _():
                pltpu.sync_copy(idx3_hbm.at[cbase + c0 + 3], idxb)

            return carry

        lax.fori_loop(0, NCH // 2, pair, 0)
        plsc.subcore_barrier()
        pltpu.sync_copy(aggr.at[pl.ds(sid * rows_per_tile, rows_per_tile)],
                        out_hbm.at[cid].at[pl.ds(sid * rows_per_tile, rows_per_tile)])

    return k2(xs, relmsg_l, idx3)


_RT = 2000


def _scale_body(x_ref, a_ref, o_ref):
    o_ref[...] = x_ref[...] * a_ref[...]


def _scale(x, a):
    return pl.pallas_call(
        _scale_body,
        grid=(N // _RT,),
        in_specs=[
            pl.BlockSpec((_RT, HID), lambda i: (i, 0)),
            pl.BlockSpec((_RT, 1), lambda i: (i, 0)),
        ],
        out_specs=pl.BlockSpec((_RT, HID), lambda i: (i, 0)),
        out_shape=jax.ShapeDtypeStruct((N, HID), F32),
    )(x, a)


def _conv_body(agg_ref, x_ref, cw_ref, cb_ref, xn_o):
    a = agg_ref[0] + agg_ref[1] + x_ref[...]
    xn_o[...] = jnp.dot(a, cw_ref[...], preferred_element_type=F32) + cb_ref[...]


def _conv_scale_body(agg_ref, x_ref, cw_ref, cb_ref, an_ref, xn_o, xs_o):
    a = agg_ref[0] + agg_ref[1] + x_ref[...]
    xn = jnp.dot(a, cw_ref[...], preferred_element_type=F32) + cb_ref[...]
    xn_o[...] = xn
    xs_o[...] = xn * an_ref[...]


def _conv(agg2, x, cw, cb):
    return pl.pallas_call(
        _conv_body,
        grid=(N // _RT,),
        in_specs=[
            pl.BlockSpec((NC, _RT, HID), lambda i: (0, i, 0)),
            pl.BlockSpec((_RT, HID), lambda i: (i, 0)),
            pl.BlockSpec((HID, HID), lambda i: (0, 0)),
            pl.BlockSpec((1, HID), lambda i: (0, 0)),
        ],
        out_specs=pl.BlockSpec((_RT, HID), lambda i: (i, 0)),
        out_shape=jax.ShapeDtypeStruct((N, HID), F32),
    )(agg2, x, cw, cb)


def _conv_scale(agg2, x, cw, cb, an):
    return pl.pallas_call(
        _conv_scale_body,
        grid=(N // _RT,),
        in_specs=[
            pl.BlockSpec((NC, _RT, HID), lambda i: (0, i, 0)),
            pl.BlockSpec((_RT, HID), lambda i: (i, 0)),
            pl.BlockSpec((HID, HID), lambda i: (0, 0)),
            pl.BlockSpec((1, HID), lambda i: (0, 0)),
            pl.BlockSpec((_RT, 1), lambda i: (i, 0)),
        ],
        out_specs=[
            pl.BlockSpec((_RT, HID), lambda i: (i, 0)),
            pl.BlockSpec((_RT, HID), lambda i: (i, 0)),
        ],
        out_shape=(
            jax.ShapeDtypeStruct((N, HID), F32),
            jax.ShapeDtypeStruct((N, HID), F32),
        ),
    )(agg2, x, cw, cb, an)


def _k4_body(x_ref, bat_ref, xnode_ref, mw_ref, mb_ref, out_ref):
    x = x_ref[...]
    oh = (bat_ref[...] == lax.broadcasted_iota(I32, (N, B), 1)).astype(F32)
    sums = lax.dot_general(oh, x, (((0,), (0,)), ((), ())),
                           preferred_element_type=F32)
    cnt = jnp.sum(oh, axis=0)
    xg = sums / jnp.maximum(cnt, 1.0)[:, None]
    cat = jnp.concatenate([xg, xnode_ref[...]], axis=1)
    out_ref[...] = jnp.sum(cat * mw_ref[...], axis=1, keepdims=True) + mb_ref[0, 0]


def _k4(x, bat2, xnode, mw, mb):
    return pl.pallas_call(
        _k4_body,
        out_shape=jax.ShapeDtypeStruct((B, 1), F32),
    )(x, bat2, xnode, mw, mb)


def kernel(node_ids, rel_ids, edge_index, batch, visit_node, ehr_nodes,
           node_emb, rel_emb, lin_W, lin_b, alpha_W, alpha_b, beta_W, beta_b,
           wr_W, wr_b, conv_W, conv_b, mlp_W, mlp_b):
    xproj, relmsg, xnode = _k0(node_emb, rel_emb, lin_W, lin_b,
                               wr_W.reshape(LAYERS, HID),
                               wr_b.reshape(LAYERS), ehr_nodes)
    attn = _k1(visit_node, alpha_W, alpha_b.reshape(LAYERS, 1, N_KG),
               beta_W.reshape(LAYERS, 1, N_KG), beta_b.reshape(LAYERS, 1, 1))

    nid_p = jnp.concatenate([node_ids.astype(I32), jnp.zeros((NP - N,), I32)])
    bat_p = jnp.concatenate([batch.astype(I32), jnp.zeros((NP - N,), I32)])
    x0p, anf = _k1b(nid_p, bat_p, xproj, attn.reshape(LAYERS * B * N_KG))
    x = x0p[:N]
    an = anf.reshape(LAYERS, NP)[:, :N]
    xs = _scale(x, an[0].reshape(N, 1))

    pads = EP - E
    srcp = jnp.concatenate([edge_index[0].astype(I32), jnp.zeros((pads,), I32)])
    dstp = jnp.concatenate([edge_index[1].astype(I32), jnp.full((pads,), N, I32)])
    ridp = jnp.concatenate([rel_ids.astype(I32), jnp.zeros((pads,), I32)])
    idx3 = jnp.stack([srcp.reshape(-1, CH), dstp.reshape(-1, CH),
                      ridp.reshape(-1, CH)], axis=1)

    for l in range(LAYERS):
        agg2 = _edge_pass(xs, relmsg[l], idx3)[:, :N]
        if l < LAYERS - 1:
            x, xs = _conv_scale(agg2, x, conv_W[l], conv_b[l].reshape(1, HID),
                                an[l + 1].reshape(N, 1))
        else:
            x = _conv(agg2, x, conv_W[l], conv_b[l].reshape(1, HID))

    return _k4(x, batch.reshape(N, 1).astype(I32), xnode,
               mlp_W.reshape(1, 2 * HID), mlp_b.reshape(1, 1))
